# baseline (speedup 1.0000x reference)
import functools

import numpy as np

import jax
import jax.numpy as jnp
from jax import lax
from jax.experimental import pallas as pl
from jax.experimental.pallas import tpu as pltpu
from jax.experimental.pallas import tpu_sc as plsc

_NC = 2
_NS = 16
_NW = _NC * _NS
_L = 16
_NBUF = 4


def _project_table(emb, W1p):
    V, E = emb.shape
    H = W1p.shape[1]
    VB = 10000

    def body(emb_ref, w1_ref, out_ref):
        out_ref[...] = jnp.dot(emb_ref[...], w1_ref[...],
                               preferred_element_type=jnp.float32
                               ).astype(jnp.bfloat16)

    return pl.pallas_call(
        body,
        grid=(V // VB,),
        in_specs=[pl.BlockSpec((VB, E), lambda i: (i, 0)),
                  pl.BlockSpec((E, H), lambda i: (0, 0))],
        out_specs=pl.BlockSpec((VB, H), lambda i: (i, 0)),
        out_shape=jax.ShapeDtypeStruct((V, H), jnp.bfloat16),
    )(emb, W1p)


def _sc_pool(table, x, B, S, H, bpw):
    mesh = plsc.VectorSubcoreMesh(core_axis_name="c", subcore_axis_name="s")
    NG = H // 32
    mask = jnp.int32(-65536)

    @functools.partial(
        pl.kernel,
        out_type=jax.ShapeDtypeStruct((B, H), jnp.float32),
        mesh=mesh,
        compiler_params=pltpu.CompilerParams(use_tc_tiling_on_sc=False,
                                             needs_layout_passes=False),
        scratch_types=[
            pltpu.VMEM((bpw, S), jnp.int32),
            pltpu.VMEM((S, bpw), jnp.int32),
            pltpu.VMEM((_NBUF, bpw, H), jnp.bfloat16),
            pltpu.VMEM((bpw, H), jnp.float32),
            [pltpu.SemaphoreType.DMA] * _NBUF,
        ],
    )
    def k(table_hbm, x_hbm, out_hbm, xraw_v, idx_v, bufs, acc_v, sems):
        w = lax.axis_index("s") * _NC + lax.axis_index("c")
        pltpu.sync_copy(x_hbm.at[pl.ds(w * bpw, bpw)], xraw_v)

        lanes = lax.iota(jnp.int32, _L)

        def trow(s, carry):
            for kk in range(bpw // _L):
                rows = lanes + (kk * _L)
                cols = jnp.full((_L,), 0, jnp.int32) + s
                v = plsc.load_gather(xraw_v, [rows, cols])
                idx_v[s, pl.ds(kk * _L, _L)] = v
            return carry

        lax.fori_loop(0, S, trow, None, unroll=2)

        zero = jnp.zeros((_L,), jnp.float32)

        def zrow(i, carry):
            for j in range(H // _L):
                acc_v[i, pl.ds(j * _L, _L)] = zero
            return carry

        lax.fori_loop(0, bpw, zrow, None, unroll=8)

        def halves(words):
            lo = plsc.bitcast(jnp.left_shift(words, 16), jnp.float32)
            hi = plsc.bitcast(jnp.bitwise_and(words, mask), jnp.float32)
            return lo, hi

        def accum_pair(buf_a, buf_b):
            def row(i, carry):
                for g in range(NG):
                    wa = plsc.bitcast(buf_a[i, pl.ds(g * 32, 32)], jnp.int32)
                    wb = plsc.bitcast(buf_b[i, pl.ds(g * 32, 32)], jnp.int32)
                    lo_a, hi_a = halves(wa)
                    lo_b, hi_b = halves(wb)
                    plsc.addupdate(acc_v.at[i, pl.ds(g * 32, _L)],
                                   lo_a + lo_b)
                    plsc.addupdate(acc_v.at[i, pl.ds(g * 32 + _L, _L)],
                                   hi_a + hi_b)
                return carry
            lax.fori_loop(0, bpw, row, None, unroll=8)

        for b in range(_NBUF):
            pltpu.async_copy(table_hbm.at[idx_v.at[b]], bufs.at[b], sems[b])

        def step(it, carry):
            s = it * _NBUF
            for p in range(_NBUF // 2):
                b0, b1 = 2 * p, 2 * p + 1
                pltpu.make_async_copy(
                    table_hbm.at[idx_v.at[s + b0]], bufs.at[b0], sems[b0]
                ).wait()
                pltpu.make_async_copy(
                    table_hbm.at[idx_v.at[s + b1]], bufs.at[b1], sems[b1]
                ).wait()
                accum_pair(bufs.at[b0], bufs.at[b1])
                for b in (b0, b1):

                    @pl.when(s + b + _NBUF < S)
                    def _():
                        pltpu.async_copy(
                            table_hbm.at[idx_v.at[s + b + _NBUF]],
                            bufs.at[b], sems[b])
            return carry

        lax.fori_loop(0, S // _NBUF, step, None)
        pltpu.sync_copy(acc_v, out_hbm.at[pl.ds(w * bpw, bpw)])

    return k(table, x)


def _mlp(pooled_sum, b1, W2, b2, S):
    B, H = pooled_sum.shape
    C = W2.shape[1]
    BB = 512
    inv_s = 1.0 / S

    def body(ps_ref, b1_ref, w2_ref, b2_ref, out_ref):
        h = ps_ref[...] * inv_s + b1_ref[...]
        h = jnp.maximum(h, 0.0)
        out_ref[...] = jnp.dot(h, w2_ref[...],
                               preferred_element_type=jnp.float32) + b2_ref[...]

    return pl.pallas_call(
        body,
        grid=(B // BB,),
        in_specs=[pl.BlockSpec((BB, H), lambda i: (i, 0)),
                  pl.BlockSpec((1, H), lambda i: (0, 0)),
                  pl.BlockSpec((H, C), lambda i: (0, 0)),
                  pl.BlockSpec((1, C), lambda i: (0, 0))],
        out_specs=pl.BlockSpec((BB, C), lambda i: (i, 0)),
        out_shape=jax.ShapeDtypeStruct((B, C), jnp.float32),
    )(pooled_sum, b1.reshape(1, H), W2, b2.reshape(1, C))


def kernel(x, emb, W1, b1, W2, b2):
    B, S = x.shape
    H = W1.shape[1]
    bpw = B // _NW
    perm = np.array([32 * (m // 32) + 16 * (m % 2) + (m % 32) // 2
                     for m in range(H)], dtype=np.int32)
    table = _project_table(emb, W1[:, perm])
    pooled_sum = _sc_pool(table, x, B, S, H, bpw)
    return _mlp(pooled_sum, b1, W2, b2, S)


# speedup vs baseline: 1.4028x; 1.4028x over previous
import functools

import jax
import jax.numpy as jnp
from jax import lax
from jax.experimental import pallas as pl
from jax.experimental.pallas import tpu as pltpu
from jax.experimental.pallas import tpu_sc as plsc

_NC = 2
_NS = 16
_NW = _NC * _NS
_L = 16
_NBUF = 4


def _project_table(emb, W1):
    V, E = emb.shape
    H = W1.shape[1]
    VB = 10000

    def body(emb_ref, w1_ref, out_ref):
        r = jnp.dot(emb_ref[...], w1_ref[...],
                    preferred_element_type=jnp.float32)
        out_ref[:, 0:H] = r
        out_ref[:, H:2 * H] = r

    return pl.pallas_call(
        body,
        grid=(V // VB,),
        in_specs=[pl.BlockSpec((VB, E), lambda i: (i, 0)),
                  pl.BlockSpec((E, H), lambda i: (0, 0))],
        out_specs=pl.BlockSpec((VB, 2 * H), lambda i: (i, 0)),
        out_shape=jax.ShapeDtypeStruct((V, 2 * H), jnp.float32),
    )(emb, W1).reshape(2 * V, H)


def _sc_pool(table, x, B, S, H, bpw):
    mesh = plsc.VectorSubcoreMesh(core_axis_name="c", subcore_axis_name="s")

    @functools.partial(
        pl.kernel,
        out_type=jax.ShapeDtypeStruct((B, H), jnp.float32),
        mesh=mesh,
        compiler_params=pltpu.CompilerParams(use_tc_tiling_on_sc=False,
                                             needs_layout_passes=False),
        scratch_types=[
            pltpu.VMEM((bpw, S), jnp.int32),
            pltpu.VMEM((S, bpw), jnp.int32),
            pltpu.VMEM((_NBUF, bpw, H), jnp.float32),
            pltpu.VMEM((bpw, H), jnp.float32),
            [pltpu.SemaphoreType.DMA] * _NBUF,
        ],
    )
    def k(table_hbm, x_hbm, out_hbm, xraw_v, idx_v, bufs, acc_v, sems):
        w = lax.axis_index("s") * _NC + lax.axis_index("c")
        pltpu.sync_copy(x_hbm.at[pl.ds(w * bpw, bpw)], xraw_v)

        lanes = lax.iota(jnp.int32, _L)

        def trow(s, carry):
            for kk in range(bpw // _L):
                rows = lanes + (kk * _L)
                cols = jnp.full((_L,), 0, jnp.int32) + s
                v = plsc.load_gather(xraw_v, [rows, cols])
                idx_v[s, pl.ds(kk * _L, _L)] = v + v
            return carry

        lax.fori_loop(0, S, trow, None, unroll=2)

        zero = jnp.zeros((_L,), jnp.float32)

        def zrow(i, carry):
            for j in range(H // _L):
                acc_v[i, pl.ds(j * _L, _L)] = zero
            return carry

        lax.fori_loop(0, bpw, zrow, None, unroll=8)

        def accum(buf):
            def row(i, carry):
                for j in range(H // _L):
                    sl = (i, pl.ds(j * _L, _L))
                    plsc.addupdate(acc_v.at[sl], buf[sl])
                return carry
            lax.fori_loop(0, bpw, row, None, unroll=8)

        for b in range(_NBUF):
            pltpu.async_copy(table_hbm.at[idx_v.at[b]], bufs.at[b], sems[b])

        def step(it, carry):
            s = it * _NBUF
            for b in range(_NBUF):
                pltpu.make_async_copy(
                    table_hbm.at[idx_v.at[s + b]], bufs.at[b], sems[b]
                ).wait()
                accum(bufs.at[b])

                @pl.when(s + b + _NBUF < S)
                def _():
                    pltpu.async_copy(
                        table_hbm.at[idx_v.at[s + b + _NBUF]],
                        bufs.at[b], sems[b])
            return carry

        lax.fori_loop(0, S // _NBUF, step, None)
        pltpu.sync_copy(acc_v, out_hbm.at[pl.ds(w * bpw, bpw)])

    return k(table, x)


def _mlp(pooled_sum, b1, W2, b2, S):
    B, H = pooled_sum.shape
    C = W2.shape[1]
    BB = 512
    inv_s = 1.0 / S

    def body(ps_ref, b1_ref, w2_ref, b2_ref, out_ref):
        h = ps_ref[...] * inv_s + b1_ref[...]
        h = jnp.maximum(h, 0.0)
        out_ref[...] = jnp.dot(h, w2_ref[...],
                               preferred_element_type=jnp.float32) + b2_ref[...]

    return pl.pallas_call(
        body,
        grid=(B // BB,),
        in_specs=[pl.BlockSpec((BB, H), lambda i: (i, 0)),
                  pl.BlockSpec((1, H), lambda i: (0, 0)),
                  pl.BlockSpec((H, C), lambda i: (0, 0)),
                  pl.BlockSpec((1, C), lambda i: (0, 0))],
        out_specs=pl.BlockSpec((BB, C), lambda i: (i, 0)),
        out_shape=jax.ShapeDtypeStruct((B, C), jnp.float32),
    )(pooled_sum, b1.reshape(1, H), W2, b2.reshape(1, C))


def kernel(x, emb, W1, b1, W2, b2):
    B, S = x.shape
    H = W1.shape[1]
    bpw = B // _NW
    table = _project_table(emb, W1)
    pooled_sum = _sc_pool(table, x, B, S, H, bpw)
    return _mlp(pooled_sum, b1, W2, b2, S)
